# Initial kernel scaffold; baseline (speedup 1.0000x reference)
import jax
import jax.numpy as jnp
from jax.experimental import pallas as pl


def kernel(x, xyz, idx, w_att, agent_w, agent_gamma, agent_beta, mom_w, mom_gamma, mom_beta):
    raise NotImplementedError("write your pallas kernel here")


# speedup vs baseline: 1.1425x; 1.1425x over previous
import functools

import jax
import jax.numpy as jnp
from jax import lax
from jax.experimental import pallas as pl
from jax.experimental.pallas import tpu as pltpu
from jax.experimental.pallas import tpu_sc as plsc

_BN, _C, _N, _K, _CN, _CL = 8, 128, 8192, 32, 256, 8
_NC = _BN * _CN
_FLAT = _BN * _N
_EPS = 1e-5

_NWORK = 32
_CPW = _NC // _NWORK
_CHUNK = 4


def _prep_body(x_ref, att_ref, idx_ref, fx_ref, adj_ref):
    b = pl.program_id(0)
    xs = x_ref[0] * att_ref[0]
    fx_ref[0] = jnp.transpose(xs, (1, 0))
    adj_ref[0] = jnp.concatenate(
        [idx_ref[0] + b * _N, jnp.zeros((_N, _C - _K), jnp.int32)], axis=-1)


@functools.cache
def _build_prep():
    return pl.pallas_call(
        _prep_body,
        grid=(_BN,),
        in_specs=[
            pl.BlockSpec((1, _C, _N), lambda b: (b, 0, 0)),
            pl.BlockSpec((1, 1, _N), lambda b: (b, 0, 0)),
            pl.BlockSpec((1, _N, _K), lambda b: (b, 0, 0)),
        ],
        out_specs=[
            pl.BlockSpec((1, _N, _C), lambda b: (b, 0, 0)),
            pl.BlockSpec((1, _N, _C), lambda b: (b, 0, 0)),
        ],
        out_shape=[
            jax.ShapeDtypeStruct((_BN, _N, _C), jnp.float32),
            jax.ShapeDtypeStruct((_BN, _N, _C), jnp.int32),
        ],
    )


def _sc_gather_body(fx_hbm, adj_hbm, cf_hbm, rows_out, pick_out, a_out,
                    cf_v, crows_v, pick_v, a_v, sem):
    wid = lax.axis_index("s") * 2 + lax.axis_index("c")
    base = wid * _CPW
    pltpu.sync_copy(cf_hbm.at[pl.ds(base, _CPW)], cf_v)
    pltpu.async_copy(fx_hbm.at[cf_v], crows_v, sem).wait()
    pltpu.sync_copy(crows_v, rows_out.at[pl.ds(base, _CPW)])
    pltpu.async_copy(adj_hbm.at[cf_v], pick_v, sem).wait()
    pltpu.sync_copy(pick_v, pick_out.at[pl.ds(base, _CPW)])

    def chunk(i, carry):
        copies = [
            pltpu.async_copy(
                fx_hbm.at[pick_v.at[i * _CHUNK + j, pl.ds(0, _K)]],
                a_v.at[j], sem)
            for j in range(_CHUNK)
        ]
        for cp in copies:
            cp.wait()
        pltpu.sync_copy(a_v, a_out.at[pl.ds(base + i * _CHUNK, _CHUNK)])
        return carry

    lax.fori_loop(0, _CPW // _CHUNK, chunk, 0)


@functools.cache
def _build_sc_gather():
    mesh = plsc.VectorSubcoreMesh(core_axis_name="c", subcore_axis_name="s")
    return pl.kernel(
        _sc_gather_body,
        mesh=mesh,
        out_type=[
            jax.ShapeDtypeStruct((_NC, _C), jnp.float32),
            jax.ShapeDtypeStruct((_NC, _C), jnp.int32),
            jax.ShapeDtypeStruct((_NC, _K, _C), jnp.float32),
        ],
        scratch_types=[
            pltpu.VMEM((_CPW,), jnp.int32),
            pltpu.VMEM((_CPW, _C), jnp.float32),
            pltpu.VMEM((_CPW, _C), jnp.int32),
            pltpu.VMEM((_CHUNK, _K, _C), jnp.float32),
            pltpu.SemaphoreType.DMA,
        ],
    )


def _sc_rows_body(fx_hbm, cf_hbm, rows_out, cf_v, crows_v, sem):
    wid = lax.axis_index("s") * 2 + lax.axis_index("c")
    base = wid * _CPW
    pltpu.sync_copy(cf_hbm.at[pl.ds(base, _CPW)], cf_v)
    pltpu.async_copy(fx_hbm.at[cf_v], crows_v, sem).wait()
    pltpu.sync_copy(crows_v, rows_out.at[pl.ds(base, _CPW)])


@functools.cache
def _build_sc_rows():
    mesh = plsc.VectorSubcoreMesh(core_axis_name="c", subcore_axis_name="s")
    return pl.kernel(
        _sc_rows_body,
        mesh=mesh,
        out_type=jax.ShapeDtypeStruct((_NC, _C), jnp.float32),
        scratch_types=[
            pltpu.VMEM((_CPW,), jnp.int32),
            pltpu.VMEM((_CPW, _C), jnp.float32),
            pltpu.SemaphoreType.DMA,
        ],
    )


def _sc_gather_call(fx, adj, cf):
    return _build_sc_gather()(fx, adj, cf)


def _sc_rows_call(fx, cf):
    return _build_sc_rows()(fx, cf)


def _bf(x):
    return x.astype(jnp.bfloat16).astype(jnp.float32)


def _mom_body(cur_ref, pre_ref, wc_ref, wp_ref, mg_ref, mb_ref, mms_ref):
    cr = cur_ref[...]
    pr = pre_ref[...]
    mm = (jnp.dot(_bf(cr), _bf(wc_ref[...]),
                  preferred_element_type=jnp.float32)
          + jnp.dot(_bf(pr), _bf(wp_ref[...]),
                    preferred_element_type=jnp.float32))
    m = jnp.mean(mm, axis=0, keepdims=True)
    v = jnp.mean((mm - m) ** 2, axis=0, keepdims=True)
    mmn = (mm - m) / jnp.sqrt(v + _EPS) * mg_ref[...] + mb_ref[...]
    mms_ref[...] = jax.nn.softmax(mmn, axis=1)


@functools.cache
def _build_mom():
    return pl.pallas_call(
        _mom_body,
        out_shape=jax.ShapeDtypeStruct((_NC, 2), jnp.float32),
    )


def _scramble_att(mms):
    ms3 = mms.reshape(_BN, _CN, 2).transpose(0, 2, 1).reshape(_BN, 2 * _CN)
    att0 = ms3[:, 0::2].reshape(_NC, 1)
    att1 = ms3[:, 1::2].reshape(_NC, 1)
    return att0, att1


def _proj_body(a_ref, cur_ref, pre_ref, a0_ref, a1_ref, w1_ref, w2_ref,
               pren_ref, l_ref, dot_ref, n2_ref, n1_ref):
    a = a_ref[...]
    cr = cur_ref[...]
    pren = cr * a0_ref[...] + pre_ref[...] * a1_ref[...]
    pren_ref[...] = pren
    curv = cr - pren
    n1_ref[...] = jnp.sqrt(jnp.sum(curv * curv, axis=1, keepdims=True))
    w1 = _bf(w1_ref[...])[None]
    lw1 = jnp.sum(_bf(a) * w1, axis=2)
    t2 = jnp.sum(_bf(pren) * _bf(w2_ref[...]), axis=1, keepdims=True)
    l_ref[...] = lw1 + t2
    neigh = a - cr[:, None, :]
    cv = _bf(curv)[:, None, :]
    dot_ref[...] = jnp.sum(cv * _bf(neigh), axis=2)
    n2_ref[...] = jnp.sqrt(jnp.sum(neigh * neigh, axis=2))


_G = 256


@functools.cache
def _build_proj():
    return pl.pallas_call(
        _proj_body,
        grid=(_NC // _G,),
        in_specs=[
            pl.BlockSpec((_G, _K, _C), lambda i: (i, 0, 0)),
            pl.BlockSpec((_G, _C), lambda i: (i, 0)),
            pl.BlockSpec((_G, _C), lambda i: (i, 0)),
            pl.BlockSpec((_G, 1), lambda i: (i, 0)),
            pl.BlockSpec((_G, 1), lambda i: (i, 0)),
            pl.BlockSpec((1, _C), lambda i: (0, 0)),
            pl.BlockSpec((1, _C), lambda i: (0, 0)),
        ],
        out_specs=[
            pl.BlockSpec((_G, _C), lambda i: (i, 0)),
            pl.BlockSpec((_G, _K), lambda i: (i, 0)),
            pl.BlockSpec((_G, _K), lambda i: (i, 0)),
            pl.BlockSpec((_G, _K), lambda i: (i, 0)),
            pl.BlockSpec((_G, 1), lambda i: (i, 0)),
        ],
        out_shape=[
            jax.ShapeDtypeStruct((_NC, _C), jnp.float32),
            jax.ShapeDtypeStruct((_NC, _K), jnp.float32),
            jax.ShapeDtypeStruct((_NC, _K), jnp.float32),
            jax.ShapeDtypeStruct((_NC, _K), jnp.float32),
            jax.ShapeDtypeStruct((_NC, 1), jnp.float32),
        ],
    )


def _sel_body(l_ref, dot_ref, n2_ref, n1_ref, pick_ref, g_ref, b_ref,
              chosen_ref):
    l = l_ref[...]
    m = jnp.mean(l)
    v = jnp.mean((l - m) ** 2)
    lb = (l - m) / jnp.sqrt(v + _EPS) * g_ref[0, 0] + b_ref[0, 0]
    div = jnp.maximum(n1_ref[...] * n2_ref[...], 1e-8)
    d = jnp.clip(1.0 + dot_ref[...] / div, 0.0, 1.0)
    lb = lb * d
    lbw = jnp.concatenate(
        [lb, jnp.full((_NC, _C - _K), -jnp.inf, lb.dtype)], axis=1)
    ks = jnp.argmax(lbw, axis=1)
    sel = lax.broadcasted_iota(jnp.int32, (_NC, _C), 1) == ks[:, None]
    chosen_ref[...] = jnp.sum(jnp.where(sel, pick_ref[...], 0), axis=1,
                              keepdims=True)


@functools.cache
def _build_sel():
    return pl.pallas_call(
        _sel_body,
        out_shape=jax.ShapeDtypeStruct((_NC, 1), jnp.int32),
    )


def kernel(x, xyz, idx, w_att, agent_w, agent_gamma, agent_beta,
           mom_w, mom_gamma, mom_beta):
    del xyz
    x_att = jax.nn.sigmoid(jnp.einsum('oc,bcn->bon', w_att[:, :, 0], x))
    _, start = lax.top_k(x_att[:, 0, :], _CN)

    fx3, adj3 = _build_prep()(x, x_att, idx)
    fx = fx3.reshape(_FLAT, _C)
    adj = adj3.reshape(_FLAT, _C)

    start_flat = (start.astype(jnp.int32)
                  + (jnp.arange(_BN, dtype=jnp.int32) * _N)[:, None]).reshape(-1)

    w1 = agent_w[0, :_C, 0, 0].reshape(1, _C)
    w2 = agent_w[0, _C:, 0, 0].reshape(1, _C)
    wc = jnp.transpose(mom_w[:, :_C, 0])
    wp = jnp.transpose(mom_w[:, _C:, 0])
    mg = mom_gamma.reshape(1, 2)
    mb = mom_beta.reshape(1, 2)
    ag = agent_gamma.reshape(1, 1)
    ab = agent_beta.reshape(1, 1)

    ones_a = jnp.ones((_NC, 1), jnp.float32)
    zeros_a = jnp.zeros((_NC, 1), jnp.float32)

    cf = start_flat
    pre = None
    outs = []
    for step in range(_CL):
        rows, pick, a_g = _sc_gather_call(fx, adj, cf)
        if step == 0:
            att0, att1, pre = ones_a, zeros_a, rows
        else:
            outs.append(rows)
            mms = _build_mom()(rows, pre, wc, wp, mg, mb)
            att0, att1 = _scramble_att(mms)
        pren, l, dt, n2, n1 = _build_proj()(a_g, rows, pre, att0, att1,
                                            w1, w2)
        chosen = _build_sel()(l, dt, n2, n1, pick, ag, ab)
        cf = chosen[:, 0]
        pre = pren
    outs.append(_sc_rows_call(fx, cf))

    out = jnp.stack(outs, axis=-1)
    out = out.reshape(_BN, _CN, _C, _CL).transpose(0, 2, 1, 3)
    return out


# speedup vs baseline: 1.3030x; 1.1404x over previous
import functools

import jax
import jax.numpy as jnp
from jax import lax
from jax.experimental import pallas as pl
from jax.experimental.pallas import tpu as pltpu
from jax.experimental.pallas import tpu_sc as plsc

_BN, _C, _N, _K, _CN, _CL = 8, 128, 8192, 32, 256, 8
_NC = _BN * _CN
_FLAT = _BN * _N
_EPS = 1e-5

_NWORK = 32
_CPW = _NC // _NWORK
_CHUNK = 4


def _prep_body(x_ref, att_ref, idx_ref, fx_ref, adj_ref):
    b = pl.program_id(0)
    xs = x_ref[0] * att_ref[0]
    fx_ref[0] = jnp.transpose(xs, (1, 0))
    adj_ref[0] = jnp.concatenate(
        [idx_ref[0] + b * _N, jnp.zeros((_N, _C - _K), jnp.int32)], axis=-1)


@functools.cache
def _build_prep():
    return pl.pallas_call(
        _prep_body,
        grid=(_BN,),
        in_specs=[
            pl.BlockSpec((1, _C, _N), lambda b: (b, 0, 0)),
            pl.BlockSpec((1, 1, _N), lambda b: (b, 0, 0)),
            pl.BlockSpec((1, _N, _K), lambda b: (b, 0, 0)),
        ],
        out_specs=[
            pl.BlockSpec((1, _N, _C), lambda b: (b, 0, 0)),
            pl.BlockSpec((1, _N, _C), lambda b: (b, 0, 0)),
        ],
        out_shape=[
            jax.ShapeDtypeStruct((_BN, _N, _C), jnp.float32),
            jax.ShapeDtypeStruct((_BN, _N, _C), jnp.int32),
        ],
    )


def _sc_gather_body(fx_hbm, adj_hbm, cf_hbm, rows_out, pick_out, a_out,
                    cf_v, crows_v, pick_v, a0_v, a1_v, sem, sem0, sem1):
    wid = lax.axis_index("s") * 2 + lax.axis_index("c")
    base = wid * _CPW
    pltpu.sync_copy(cf_hbm.at[pl.ds(base, _CPW)], cf_v)
    pltpu.async_copy(fx_hbm.at[cf_v], crows_v, sem).wait()
    pltpu.sync_copy(crows_v, rows_out.at[pl.ds(base, _CPW)])
    pltpu.async_copy(adj_hbm.at[cf_v], pick_v, sem).wait()
    pltpu.sync_copy(pick_v, pick_out.at[pl.ds(base, _CPW)])

    def issue(ci, buf, sm):
        for j in range(_CHUNK):
            pltpu.async_copy(
                fx_hbm.at[pick_v.at[ci * _CHUNK + j, pl.ds(0, _K)]],
                buf.at[j], sm)

    def drain_wb(ci, buf, sm):
        for j in range(_CHUNK):
            pltpu.make_async_copy(
                fx_hbm.at[pick_v.at[0, pl.ds(0, _K)]], buf.at[j], sm).wait()
        pltpu.sync_copy(buf, a_out.at[pl.ds(base + ci * _CHUNK, _CHUNK)])

    nchunks = _CPW // _CHUNK
    issue(0, a0_v, sem0)

    def outer(i, carry):
        issue(2 * i + 1, a1_v, sem1)
        drain_wb(2 * i, a0_v, sem0)

        @pl.when(i < nchunks // 2 - 1)
        def _():
            issue(2 * i + 2, a0_v, sem0)

        drain_wb(2 * i + 1, a1_v, sem1)
        return carry

    lax.fori_loop(0, nchunks // 2, outer, 0)


@functools.cache
def _build_sc_gather():
    mesh = plsc.VectorSubcoreMesh(core_axis_name="c", subcore_axis_name="s")
    return pl.kernel(
        _sc_gather_body,
        mesh=mesh,
        out_type=[
            jax.ShapeDtypeStruct((_NC, _C), jnp.float32),
            jax.ShapeDtypeStruct((_NC, _C), jnp.int32),
            jax.ShapeDtypeStruct((_NC, _K, _C), jnp.float32),
        ],
        scratch_types=[
            pltpu.VMEM((_CPW,), jnp.int32),
            pltpu.VMEM((_CPW, _C), jnp.float32),
            pltpu.VMEM((_CPW, _C), jnp.int32),
            pltpu.VMEM((_CHUNK, _K, _C), jnp.float32),
            pltpu.VMEM((_CHUNK, _K, _C), jnp.float32),
            pltpu.SemaphoreType.DMA,
            pltpu.SemaphoreType.DMA,
            pltpu.SemaphoreType.DMA,
        ],
    )


def _sc_rows_body(fx_hbm, cf_hbm, rows_out, cf_v, crows_v, sem):
    wid = lax.axis_index("s") * 2 + lax.axis_index("c")
    base = wid * _CPW
    pltpu.sync_copy(cf_hbm.at[pl.ds(base, _CPW)], cf_v)
    pltpu.async_copy(fx_hbm.at[cf_v], crows_v, sem).wait()
    pltpu.sync_copy(crows_v, rows_out.at[pl.ds(base, _CPW)])


@functools.cache
def _build_sc_rows():
    mesh = plsc.VectorSubcoreMesh(core_axis_name="c", subcore_axis_name="s")
    return pl.kernel(
        _sc_rows_body,
        mesh=mesh,
        out_type=jax.ShapeDtypeStruct((_NC, _C), jnp.float32),
        scratch_types=[
            pltpu.VMEM((_CPW,), jnp.int32),
            pltpu.VMEM((_CPW, _C), jnp.float32),
            pltpu.SemaphoreType.DMA,
        ],
    )


def _sc_gather_call(fx, adj, cf):
    return _build_sc_gather()(fx, adj, cf)


def _sc_rows_call(fx, cf):
    return _build_sc_rows()(fx, cf)


def _bf(x):
    return x.astype(jnp.bfloat16).astype(jnp.float32)


def _mom_body(cur_ref, pre_ref, wm_ref, mg_ref, mb_ref, mms_ref):
    cat = jnp.concatenate([cur_ref[...], pre_ref[...]], axis=1)
    mm = jnp.dot(_bf(cat), _bf(wm_ref[...]),
                 preferred_element_type=jnp.float32)
    m = jnp.mean(mm, axis=0, keepdims=True)
    v = jnp.mean((mm - m) ** 2, axis=0, keepdims=True)
    mmn = (mm - m) / jnp.sqrt(v + _EPS) * mg_ref[...] + mb_ref[...]
    mms_ref[...] = jax.nn.softmax(mmn, axis=1)


@functools.cache
def _build_mom():
    return pl.pallas_call(
        _mom_body,
        out_shape=jax.ShapeDtypeStruct((_NC, 2), jnp.float32),
    )


def _scramble_att(mms):
    ms3 = mms.reshape(_BN, _CN, 2).transpose(0, 2, 1).reshape(_BN, 2 * _CN)
    att0 = ms3[:, 0::2].reshape(_NC, 1)
    att1 = ms3[:, 1::2].reshape(_NC, 1)
    return att0, att1


def _proj_body(a_ref, cur_ref, pre_ref, a0_ref, a1_ref, wlg_ref,
               pren_ref, l_ref, dot_ref, n2_ref, n1_ref):
    a = a_ref[...]
    cr = cur_ref[...]
    pren = cr * a0_ref[...] + pre_ref[...] * a1_ref[...]
    pren_ref[...] = pren
    curv = cr - pren
    n1_ref[...] = jnp.sqrt(jnp.sum(curv * curv, axis=1, keepdims=True))
    af = _bf(a).reshape(_G * _K, _C)
    pf = jnp.broadcast_to(_bf(pren)[:, None, :], (_G, _K, _C))
    cat = jnp.concatenate([af, pf.reshape(_G * _K, _C)], axis=1)
    l_ref[...] = jnp.dot(cat, _bf(wlg_ref[...]),
                         preferred_element_type=jnp.float32).reshape(_G, _K)
    neigh = a - cr[:, None, :]
    cv = _bf(curv)[:, None, :]
    dot_ref[...] = jnp.sum(cv * _bf(neigh), axis=2)
    n2_ref[...] = jnp.sqrt(jnp.sum(neigh * neigh, axis=2))


_G = 256


@functools.cache
def _build_proj():
    return pl.pallas_call(
        _proj_body,
        grid=(_NC // _G,),
        in_specs=[
            pl.BlockSpec((_G, _K, _C), lambda i: (i, 0, 0)),
            pl.BlockSpec((_G, _C), lambda i: (i, 0)),
            pl.BlockSpec((_G, _C), lambda i: (i, 0)),
            pl.BlockSpec((_G, 1), lambda i: (i, 0)),
            pl.BlockSpec((_G, 1), lambda i: (i, 0)),
            pl.BlockSpec((2 * _C, 1), lambda i: (0, 0)),
        ],
        out_specs=[
            pl.BlockSpec((_G, _C), lambda i: (i, 0)),
            pl.BlockSpec((_G, _K), lambda i: (i, 0)),
            pl.BlockSpec((_G, _K), lambda i: (i, 0)),
            pl.BlockSpec((_G, _K), lambda i: (i, 0)),
            pl.BlockSpec((_G, 1), lambda i: (i, 0)),
        ],
        out_shape=[
            jax.ShapeDtypeStruct((_NC, _C), jnp.float32),
            jax.ShapeDtypeStruct((_NC, _K), jnp.float32),
            jax.ShapeDtypeStruct((_NC, _K), jnp.float32),
            jax.ShapeDtypeStruct((_NC, _K), jnp.float32),
            jax.ShapeDtypeStruct((_NC, 1), jnp.float32),
        ],
    )


def _sel_body(l_ref, dot_ref, n2_ref, n1_ref, pick_ref, g_ref, b_ref,
              chosen_ref):
    l = l_ref[...]
    m = jnp.mean(l)
    v = jnp.mean((l - m) ** 2)
    lb = (l - m) / jnp.sqrt(v + _EPS) * g_ref[0, 0] + b_ref[0, 0]
    div = jnp.maximum(n1_ref[...] * n2_ref[...], 1e-8)
    d = jnp.clip(1.0 + dot_ref[...] / div, 0.0, 1.0)
    lb = lb * d
    lbw = jnp.concatenate(
        [lb, jnp.full((_NC, _C - _K), -jnp.inf, lb.dtype)], axis=1)
    ks = jnp.argmax(lbw, axis=1)
    sel = lax.broadcasted_iota(jnp.int32, (_NC, _C), 1) == ks[:, None]
    chosen_ref[...] = jnp.sum(jnp.where(sel, pick_ref[...], 0), axis=1,
                              keepdims=True)


@functools.cache
def _build_sel():
    return pl.pallas_call(
        _sel_body,
        out_shape=jax.ShapeDtypeStruct((_NC, 1), jnp.int32),
    )


def kernel(x, xyz, idx, w_att, agent_w, agent_gamma, agent_beta,
           mom_w, mom_gamma, mom_beta):
    del xyz
    x_att = jax.nn.sigmoid(jnp.einsum('oc,bcn->bon', w_att[:, :, 0], x))
    _, start = lax.top_k(x_att[:, 0, :], _CN)

    fx3, adj3 = _build_prep()(x, x_att, idx)
    fx = fx3.reshape(_FLAT, _C)
    adj = adj3.reshape(_FLAT, _C)

    start_flat = (start.astype(jnp.int32)
                  + (jnp.arange(_BN, dtype=jnp.int32) * _N)[:, None]).reshape(-1)

    wlg = agent_w[0, :, 0, 0].reshape(2 * _C, 1)
    wm = jnp.transpose(mom_w[:, :, 0])
    mg = mom_gamma.reshape(1, 2)
    mb = mom_beta.reshape(1, 2)
    ag = agent_gamma.reshape(1, 1)
    ab = agent_beta.reshape(1, 1)

    ones_a = jnp.ones((_NC, 1), jnp.float32)
    zeros_a = jnp.zeros((_NC, 1), jnp.float32)

    cf = start_flat
    pre = None
    outs = []
    for step in range(_CL):
        rows, pick, a_g = _sc_gather_call(fx, adj, cf)
        if step == 0:
            att0, att1, pre = ones_a, zeros_a, rows
        else:
            outs.append(rows)
            mms = _build_mom()(rows, pre, wm, mg, mb)
            att0, att1 = _scramble_att(mms)
        pren, l, dt, n2, n1 = _build_proj()(a_g, rows, pre, att0, att1, wlg)
        chosen = _build_sel()(l, dt, n2, n1, pick, ag, ab)
        cf = chosen[:, 0]
        pre = pren
    outs.append(_sc_rows_call(fx, cf))

    out = jnp.stack(outs, axis=-1)
    out = out.reshape(_BN, _CN, _C, _CL).transpose(0, 2, 1, 3)
    return out
